# Initial kernel scaffold; baseline (speedup 1.0000x reference)
import jax
import jax.numpy as jnp
from jax.experimental import pallas as pl


def kernel(x, t, pe):
    raise NotImplementedError("write your pallas kernel here")


# speedup vs baseline: 1.7832x; 1.7832x over previous
import functools

import jax
import jax.numpy as jnp
from jax import lax
from jax.experimental import pallas as pl
from jax.experimental.pallas import tpu as pltpu
from jax.experimental.pallas import tpu_sc as plsc

D_MODEL = 128
BATCH = 16384
LANES = 16

_info = plsc.get_sparse_core_info()
NUM_CORES = _info.num_cores
NUM_SUBCORES = _info.num_subcores
NW = NUM_CORES * NUM_SUBCORES
BPW = BATCH // NW
CHUNK = 128
NCHUNK = BPW // CHUNK


def _body(x_hbm, t_hbm, pe_hbm, out_hbm, idx_v, pe_v, x_v, gsem, xsem):
    wid = lax.axis_index("s") * NUM_CORES + lax.axis_index("c")
    base = wid * BPW
    pltpu.sync_copy(t_hbm.at[pl.ds(base, BPW)], idx_v)
    for ci in range(NCHUNK):
        off = base + ci * CHUNK
        cg = pltpu.async_copy(
            pe_hbm.at[idx_v.at[pl.ds(ci * CHUNK, CHUNK)]], pe_v, gsem
        )
        cx = pltpu.async_copy(x_hbm.at[pl.ds(off, CHUNK)], x_v, xsem)
        cg.wait()
        cx.wait()

        def row(r, carry):
            for j in range(D_MODEL // LANES):
                sl = pl.ds(j * LANES, LANES)
                x_v[r, sl] = x_v[r, sl] + pe_v[r, sl]
            return carry

        lax.fori_loop(0, CHUNK, row, 0)
        pltpu.sync_copy(x_v, out_hbm.at[pl.ds(off, CHUNK)])


@functools.partial(jax.jit, static_argnames=())
def _run(x, t, pe2d):
    mesh = plsc.VectorSubcoreMesh(core_axis_name="c", subcore_axis_name="s")
    k = pl.kernel(
        _body,
        mesh=mesh,
        out_type=jax.ShapeDtypeStruct((BATCH, D_MODEL), jnp.float32),
        scratch_types=[
            pltpu.VMEM((BPW,), jnp.int32),
            pltpu.VMEM((CHUNK, D_MODEL), jnp.float32),
            pltpu.VMEM((CHUNK, D_MODEL), jnp.float32),
            pltpu.SemaphoreType.DMA,
            pltpu.SemaphoreType.DMA,
        ],
    )
    return k(x, t, pe2d)


def kernel(x, t, pe):
    out = _run(x, t.astype(jnp.int32), pe.reshape(pe.shape[1], pe.shape[2]))
    return out[None]


# speedup vs baseline: 2.0025x; 1.1230x over previous
import functools

import jax
import jax.numpy as jnp
from jax import lax
from jax.experimental import pallas as pl
from jax.experimental.pallas import tpu as pltpu
from jax.experimental.pallas import tpu_sc as plsc

D_MODEL = 128
BATCH = 16384
LANES = 16

_info = plsc.get_sparse_core_info()
NUM_CORES = _info.num_cores
NUM_SUBCORES = _info.num_subcores
NW = NUM_CORES * NUM_SUBCORES
BPW = BATCH // NW
CHUNK = 128
NCHUNK = BPW // CHUNK


def _body(x_hbm, t_hbm, pe_hbm, out_hbm,
          idx_v, x_big, pe_v0, pe_v1, xsem, gsem0, gsem1, osem):
    wid = lax.axis_index("s") * NUM_CORES + lax.axis_index("c")
    base = wid * BPW
    pltpu.sync_copy(t_hbm.at[pl.ds(base, BPW)], idx_v)
    cx = pltpu.async_copy(x_hbm.at[pl.ds(base, BPW)], x_big, xsem)

    pe_bufs = (pe_v0, pe_v1)
    gsems = (gsem0, gsem1)
    copies = [None] * NCHUNK
    copies[0] = pltpu.async_copy(
        pe_hbm.at[idx_v.at[pl.ds(0, CHUNK)]], pe_v0, gsem0)
    copies[1] = pltpu.async_copy(
        pe_hbm.at[idx_v.at[pl.ds(CHUNK, CHUNK)]], pe_v1, gsem1)
    cx.wait()

    stores = []
    for ci in range(NCHUNK):
        cur = ci & 1
        copies[ci].wait()
        pe_b = pe_bufs[cur]

        def row(r, carry):
            for j in range(D_MODEL // LANES):
                sl = pl.ds(j * LANES, LANES)
                x_big[ci * CHUNK + r, sl] = (
                    x_big[ci * CHUNK + r, sl] + pe_b[r, sl])
            return carry

        lax.fori_loop(0, CHUNK, row, 0)
        if ci + 2 < NCHUNK:
            copies[ci + 2] = pltpu.async_copy(
                pe_hbm.at[idx_v.at[pl.ds((ci + 2) * CHUNK, CHUNK)]],
                pe_bufs[cur], gsems[cur])
        stores.append(pltpu.async_copy(
            x_big.at[pl.ds(ci * CHUNK, CHUNK)],
            out_hbm.at[pl.ds(base + ci * CHUNK, CHUNK)], osem))
    for s in stores:
        s.wait()


@jax.jit
def _run(x, t, pe2d):
    mesh = plsc.VectorSubcoreMesh(core_axis_name="c", subcore_axis_name="s")
    k = pl.kernel(
        _body,
        mesh=mesh,
        out_type=jax.ShapeDtypeStruct((BATCH, D_MODEL), jnp.float32),
        scratch_types=[
            pltpu.VMEM((BPW,), jnp.int32),
            pltpu.VMEM((BPW, D_MODEL), jnp.float32),
            pltpu.VMEM((CHUNK, D_MODEL), jnp.float32),
            pltpu.VMEM((CHUNK, D_MODEL), jnp.float32),
            pltpu.SemaphoreType.DMA,
            pltpu.SemaphoreType.DMA,
            pltpu.SemaphoreType.DMA,
            pltpu.SemaphoreType.DMA,
        ],
    )
    return k(x, t, pe2d)


def kernel(x, t, pe):
    out = _run(x, t.astype(jnp.int32), pe.reshape(pe.shape[1], pe.shape[2]))
    return out[None]
